# baseline (speedup 1.0000x reference)
import functools

import jax
import jax.numpy as jnp
import numpy as np
from jax.experimental import pallas as pl
from jax.experimental.pallas import tpu as pltpu

D_IN = 768
E = 8
HIDDEN = 256
N_TOK = 4096
EPS = 0.1
BLK = 512
GRID = N_TOK // BLK
_SQRT_HALF = 0.7071067811865476


def _moe_kernel(x_ref, wr_ref, br_ref, w1_ref, b1_ref, w2_ref, b2_ref,
                out_ref, aux_ref, load_acc):
    g = pl.program_id(0)
    x = x_ref[...]
    logits = jnp.dot(x, wr_ref[...], preferred_element_type=jnp.float32)
    logits = logits + br_ref[...]
    m = jnp.max(logits, axis=-1, keepdims=True)
    ex = jnp.exp(logits - m)
    probs = ex / jnp.sum(ex, axis=-1, keepdims=True)
    probs = (1.0 - EPS) * probs + EPS / E

    iota = jax.lax.broadcasted_iota(jnp.int32, probs.shape, 1)
    m1 = jnp.max(probs, axis=-1, keepdims=True)
    e1 = jnp.min(jnp.where(probs == m1, iota, E), axis=-1, keepdims=True)
    probs2 = jnp.where(iota == e1, -jnp.inf, probs)
    m2 = jnp.max(probs2, axis=-1, keepdims=True)
    e2 = jnp.min(jnp.where(probs2 == m2, iota, E), axis=-1, keepdims=True)
    gate = jnp.where(iota == e1, m1, 0.0) + jnp.where(iota == e2, m2, 0.0)

    psum = jnp.sum(probs, axis=0, keepdims=True)

    @pl.when(g == 0)
    def _():
        load_acc[...] = psum

    @pl.when(g != 0)
    def _():
        load_acc[...] = load_acc[...] + psum

    @pl.when(g == GRID - 1)
    def _():
        load = load_acc[...] / N_TOK
        aux = jnp.sum(load * jnp.log(load * E + 1e-9)) / np.log(E + 1e-9)
        aux_ref[...] = jnp.reshape(aux, (1, 1))

    xb = x.astype(jnp.bfloat16)
    hs = []
    for e in range(E):
        he = jnp.dot(xb, w1_ref[e], preferred_element_type=jnp.float32)
        hs.append(he)
    h = jnp.concatenate(hs, axis=1)
    h = h + b1_ref[...]
    h = 0.5 * h * (1.0 + jax.lax.erf(h * _SQRT_HALF))
    gate_exp = jnp.broadcast_to(gate[:, :, None], (BLK, E, HIDDEN))
    gate_exp = gate_exp.reshape(BLK, E * HIDDEN)
    h = (h * gate_exp).astype(jnp.bfloat16)
    y = jnp.dot(h, w2_ref[...], preferred_element_type=jnp.float32)
    y = y + jnp.dot(gate, b2_ref[...], preferred_element_type=jnp.float32)
    out_ref[...] = y


@jax.jit
def kernel(x, Wr, br, W1, b1, W2, b2):
    b1r = b1.reshape(1, E * HIDDEN)
    w1b = W1.astype(jnp.bfloat16)
    w2r = W2.reshape(E * HIDDEN, D_IN).astype(jnp.bfloat16)
    out, aux = pl.pallas_call(
        _moe_kernel,
        grid=(GRID,),
        in_specs=[
            pl.BlockSpec((BLK, D_IN), lambda g: (g, 0)),
            pl.BlockSpec((D_IN, E), lambda g: (0, 0)),
            pl.BlockSpec((E,), lambda g: (0,)),
            pl.BlockSpec((E, D_IN, HIDDEN), lambda g: (0, 0, 0)),
            pl.BlockSpec((1, E * HIDDEN), lambda g: (0, 0)),
            pl.BlockSpec((E * HIDDEN, D_IN), lambda g: (0, 0)),
            pl.BlockSpec((E, D_IN), lambda g: (0, 0)),
        ],
        out_specs=[
            pl.BlockSpec((BLK, D_IN), lambda g: (g, 0)),
            pl.BlockSpec((1, 1), lambda g: (0, 0)),
        ],
        out_shape=[
            jax.ShapeDtypeStruct((N_TOK, D_IN), jnp.float32),
            jax.ShapeDtypeStruct((1, 1), jnp.float32),
        ],
        scratch_shapes=[pltpu.VMEM((1, E), jnp.float32)],
    )(x, Wr, br, w1b, b1r, w2r, b2)
    return out, aux.reshape(())


# speedup vs baseline: 1.1506x; 1.1506x over previous
import functools

import jax
import jax.numpy as jnp
import numpy as np
from jax.experimental import pallas as pl
from jax.experimental.pallas import tpu as pltpu

D_IN = 768
E = 8
HIDDEN = 256
N_TOK = 4096
EPS = 0.1
BLK = 1024
GRID = N_TOK // BLK
_SQRT_HALF = 0.7071067811865476


def _gelu_exact(h):
    return 0.5 * h * (1.0 + jax.lax.erf(h * _SQRT_HALF))


def _moe_kernel(x_ref, wr_ref, br_ref, w1_ref, b1_ref, w2_ref, b2_ref,
                out_ref, aux_ref, load_acc):
    g = pl.program_id(0)
    x = x_ref[...]
    logits = jnp.dot(x, wr_ref[...], preferred_element_type=jnp.float32)
    logits = logits + br_ref[...]
    m = jnp.max(logits, axis=-1, keepdims=True)
    ex = jnp.exp(logits - m)
    probs = ex / jnp.sum(ex, axis=-1, keepdims=True)
    probs = (1.0 - EPS) * probs + EPS / E

    iota = jax.lax.broadcasted_iota(jnp.int32, probs.shape, 1)
    m1 = jnp.max(probs, axis=-1, keepdims=True)
    e1 = jnp.min(jnp.where(probs == m1, iota, E), axis=-1, keepdims=True)
    probs2 = jnp.where(iota == e1, -jnp.inf, probs)
    m2 = jnp.max(probs2, axis=-1, keepdims=True)
    e2 = jnp.min(jnp.where(probs2 == m2, iota, E), axis=-1, keepdims=True)

    psum = jnp.sum(probs, axis=0, keepdims=True)

    @pl.when(g == 0)
    def _():
        load_acc[...] = psum

    @pl.when(g != 0)
    def _():
        load_acc[...] = load_acc[...] + psum

    @pl.when(g == GRID - 1)
    def _():
        load = load_acc[...] / N_TOK
        aux = jnp.sum(load * jnp.log(load * E + 1e-9)) / np.log(E + 1e-9)
        aux_ref[...] = jnp.reshape(aux, (1, 1))

    acc = jnp.zeros((BLK, D_IN), jnp.float32)
    for e in range(E):
        h = jnp.dot(x, w1_ref[e], preferred_element_type=jnp.float32)
        h = _gelu_exact(h + b1_ref[e][None, :])
        y = jnp.dot(h, w2_ref[e], preferred_element_type=jnp.float32)
        y = y + b2_ref[e][None, :]
        gate = (jnp.where(e1 == e, m1, 0.0) + jnp.where(e2 == e, m2, 0.0))
        acc = acc + gate * y
    out_ref[...] = acc


@jax.jit
def kernel(x, Wr, br, W1, b1, W2, b2):
    out, aux = pl.pallas_call(
        _moe_kernel,
        grid=(GRID,),
        in_specs=[
            pl.BlockSpec((BLK, D_IN), lambda g: (g, 0)),
            pl.BlockSpec((D_IN, E), lambda g: (0, 0)),
            pl.BlockSpec((E,), lambda g: (0,)),
            pl.BlockSpec((E, D_IN, HIDDEN), lambda g: (0, 0, 0)),
            pl.BlockSpec((E, HIDDEN), lambda g: (0, 0)),
            pl.BlockSpec((E, HIDDEN, D_IN), lambda g: (0, 0, 0)),
            pl.BlockSpec((E, D_IN), lambda g: (0, 0)),
        ],
        out_specs=[
            pl.BlockSpec((BLK, D_IN), lambda g: (g, 0)),
            pl.BlockSpec((1, 1), lambda g: (0, 0)),
        ],
        out_shape=[
            jax.ShapeDtypeStruct((N_TOK, D_IN), jnp.float32),
            jax.ShapeDtypeStruct((1, 1), jnp.float32),
        ],
        scratch_shapes=[pltpu.VMEM((1, E), jnp.float32)],
    )(x, Wr, br, W1, b1, W2, b2)
    return out, aux.reshape(())
